# baseline (speedup 1.0000x reference)
import jax
import jax.numpy as jnp
from jax.experimental import pallas as pl
from jax.experimental.pallas import tpu as pltpu

_ROWS = 16384
_COLS = 4096
_NBUF = 8
_B = 256
_NSTEPS = _ROWS // _B


def _in_copy(x_ref, bufs, insem, i):
    s = i % _NBUF
    return pltpu.make_async_copy(
        x_ref.at[pl.ds(i * _B, _B), :], bufs.at[s], insem.at[s]
    )


def _out_copy(o_ref, bufs, outsem, j):
    t = j % _NBUF
    return pltpu.make_async_copy(
        bufs.at[t], o_ref.at[pl.ds(j * _B, _B), :], outsem.at[t]
    )


def _body(i1_ref, i2_ref, v_ref, x_ref, o_ref, bufs, rowbuf, insem, outsem,
          rowsem):
    lead = _NBUF - 1
    for i in range(_NSTEPS):
        if i >= _NBUF:
            _out_copy(o_ref, bufs, outsem, i - _NBUF).wait()
        _in_copy(x_ref, bufs, insem, i).start()
        j = i - lead
        if j >= 0:
            _in_copy(x_ref, bufs, insem, j).wait()
            _out_copy(o_ref, bufs, outsem, j).start()
    for j in range(_NSTEPS - lead, _NSTEPS):
        _in_copy(x_ref, bufs, insem, j).wait()
        _out_copy(o_ref, bufs, outsem, j).start()
    for j in range(_NSTEPS - _NBUF, _NSTEPS):
        _out_copy(o_ref, bufs, outsem, j).wait()

    row = i1_ref[0]
    col = i2_ref[0]
    fetch = pltpu.make_async_copy(
        x_ref.at[pl.ds(row, 1), :], rowbuf, rowsem
    )
    fetch.start()
    fetch.wait()
    lane = jax.lax.broadcasted_iota(jnp.int32, (1, _COLS), 1)
    rowbuf[...] = jnp.where(lane == col, v_ref[0], rowbuf[...])
    put = pltpu.make_async_copy(
        rowbuf, o_ref.at[pl.ds(row, 1), :], rowsem
    )
    put.start()
    put.wait()


def kernel(input, index1, index2, value):
    i1 = index1.astype(jnp.int32)
    i2 = index2.astype(jnp.int32)
    v = value.astype(jnp.float32)
    return pl.pallas_call(
        _body,
        in_specs=[
            pl.BlockSpec(memory_space=pltpu.SMEM),
            pl.BlockSpec(memory_space=pltpu.SMEM),
            pl.BlockSpec(memory_space=pltpu.SMEM),
            pl.BlockSpec(memory_space=pl.ANY),
        ],
        out_specs=pl.BlockSpec(memory_space=pl.ANY),
        out_shape=jax.ShapeDtypeStruct((_ROWS, _COLS), jnp.float32),
        scratch_shapes=[
            pltpu.VMEM((_NBUF, _B, _COLS), jnp.float32),
            pltpu.VMEM((1, _COLS), jnp.float32),
            pltpu.SemaphoreType.DMA((_NBUF,)),
            pltpu.SemaphoreType.DMA((_NBUF,)),
            pltpu.SemaphoreType.DMA,
        ],
    )(i1, i2, v, input)


# speedup vs baseline: 1.0274x; 1.0274x over previous
import jax
import jax.numpy as jnp
from jax.experimental import pallas as pl
from jax.experimental.pallas import tpu as pltpu

_ROWS = 16384
_COLS = 4096
_BLOCK_R = 960


def _body(i1_ref, i2_ref, v_ref, x_ref, o_ref):
    i = pl.program_id(0)
    o_ref[...] = x_ref[...]
    row = i1_ref[0]
    col = i2_ref[0]
    blk_start = i * _BLOCK_R

    @pl.when((row >= blk_start) & (row < blk_start + _BLOCK_R))
    def _():
        r = row - blk_start
        row_vals = x_ref[pl.ds(r, 1), :]
        lane = jax.lax.broadcasted_iota(jnp.int32, (1, _COLS), 1)
        o_ref[pl.ds(r, 1), :] = jnp.where(lane == col, v_ref[0], row_vals)


def kernel(input, index1, index2, value):
    i1 = index1.astype(jnp.int32)
    i2 = index2.astype(jnp.int32)
    v = value.astype(jnp.float32)
    return pl.pallas_call(
        _body,
        grid=(pl.cdiv(_ROWS, _BLOCK_R),),
        in_specs=[
            pl.BlockSpec(memory_space=pltpu.SMEM),
            pl.BlockSpec(memory_space=pltpu.SMEM),
            pl.BlockSpec(memory_space=pltpu.SMEM),
            pl.BlockSpec((_BLOCK_R, _COLS), lambda i: (i, 0)),
        ],
        out_specs=pl.BlockSpec((_BLOCK_R, _COLS), lambda i: (i, 0)),
        out_shape=jax.ShapeDtypeStruct((_ROWS, _COLS), jnp.float32),
        compiler_params=pltpu.CompilerParams(
            dimension_semantics=("arbitrary",),
            vmem_limit_bytes=130 * 1024 * 1024,
        ),
    )(i1, i2, v, input)
